# baseline (speedup 1.0000x reference)
import functools

import jax
import jax.numpy as jnp
from jax import lax
from jax.experimental import pallas as pl
from jax.experimental.pallas import tpu as pltpu
from jax.experimental.pallas import tpu_sc as plsc

_V = 1_000_000
_D = 32
_B = 16384
_L = 50
_FLAT = _B * _L
_LINES = _FLAT // 4

_NW = 32
_WL = _LINES // _NW
_CL = 160
_NCH = _WL // _CL
_NBUF = 4
_NGRP = _NCH // _NBUF


@functools.cache
def _build_gather():
    mesh = plsc.VectorSubcoreMesh(core_axis_name="c", subcore_axis_name="s")

    @functools.partial(
        pl.kernel,
        mesh=mesh,
        compiler_params=pltpu.CompilerParams(use_tc_tiling_on_sc=False),
        out_type=jax.ShapeDtypeStruct((_LINES, 4 * _D), jnp.float32),
        scratch_types=[
            [pltpu.VMEM((_WL,), jnp.int32) for _ in range(4)],
            [[pltpu.VMEM((_CL, _D), jnp.float32) for _ in range(4)]
             for _ in range(_NBUF)],
            [pltpu.SemaphoreType.DMA for _ in range(_NBUF)],
        ],
    )
    def _gather(tab_hbm, idx_hbm, out_hbm, idxq, rows, sems):
        wid = lax.axis_index("s") * 2 + lax.axis_index("c")
        lbase = wid * _WL
        for q in range(4):
            pltpu.sync_copy(
                idx_hbm.at[pl.ds(q * _LINES + lbase, _WL)], idxq[q]
            )

        def start(c, j):
            for q in range(4):
                pltpu.async_copy(
                    tab_hbm.at[idxq[q].at[pl.ds(c * _CL, _CL)]],
                    rows[j][q],
                    sems[j],
                )

        def drain(j):
            for q in range(4):
                pltpu.make_async_copy(
                    tab_hbm.at[idxq[0].at[pl.ds(0, _CL)]],
                    rows[j][q],
                    sems[j],
                ).wait()

        for j in range(_NBUF):
            start(j, j)

        def outer(g, carry):
            for j in range(_NBUF):
                c = g * _NBUF + j
                drain(j)
                for q in range(4):
                    pltpu.sync_copy(
                        rows[j][q],
                        out_hbm.at[
                            pl.ds(lbase + c * _CL, _CL), pl.ds(q * _D, _D)
                        ],
                    )
                nc = c + _NBUF

                @pl.when(nc < _NCH)
                def _():
                    start(nc, j)
            return carry

        lax.fori_loop(0, _NGRP, outer, 0)

    return _gather


_BB = 128


def _decode_body(r_ref, wbd_ref, b_ref, o_ref):
    y = (
        jnp.dot(r_ref[...], wbd_ref[...], preferred_element_type=jnp.float32)
        + b_ref[...]
    )
    n = y.shape[0]
    parts = [y[:, q * _D:(q + 1) * _D].reshape(n, 1, _D) for q in range(4)]
    z = jnp.concatenate(parts, axis=1)
    o_ref[...] = z.reshape(_BB, _L, _D)

def _decode(packed, Wbd, b128):
    nl = _BB * _L // 4
    return pl.pallas_call(
        _decode_body,
        grid=(_B // _BB,),
        in_specs=[
            pl.BlockSpec((nl, 4 * _D), lambda i: (i, 0)),
            pl.BlockSpec((4 * _D, 4 * _D), lambda i: (0, 0)),
            pl.BlockSpec((1, 4 * _D), lambda i: (0, 0)),
        ],
        out_specs=pl.BlockSpec((_BB, _L, _D), lambda i: (i, 0, 0)),
        out_shape=jax.ShapeDtypeStruct((_B, _L, _D), jnp.float32),
    )(packed, Wbd, b128)


def kernel(idx, table, W, b):
    idxf = idx.reshape(-1)
    idxp = jnp.reshape(jnp.transpose(jnp.reshape(idxf, (_LINES, 4))), (-1,))
    packed = _build_gather()(table, idxp)
    eye4 = jnp.eye(4, dtype=jnp.float32)
    Wbd = jnp.einsum("pq,do->pdqo", eye4, W.T).reshape(4 * _D, 4 * _D)
    b128 = jnp.tile(b, 4).reshape(1, 4 * _D)
    return _decode(packed, Wbd, b128)


# speedup vs baseline: 1.2612x; 1.2612x over previous
import functools

import jax
import jax.numpy as jnp
from jax import lax
from jax.experimental import pallas as pl
from jax.experimental.pallas import tpu as pltpu
from jax.experimental.pallas import tpu_sc as plsc

_V = 1_000_000
_D = 32
_B = 16384
_L = 50
_FLAT = _B * _L

_NW = 32
_PERW = _FLAT // _NW
_CH = 640
_NCH = _PERW // _CH
_NBUF = 4
_NGRP = _NCH // _NBUF


@functools.cache
def _build_gather():
    mesh = plsc.VectorSubcoreMesh(core_axis_name="c", subcore_axis_name="s")

    @functools.partial(
        pl.kernel,
        mesh=mesh,
        compiler_params=pltpu.CompilerParams(use_tc_tiling_on_sc=False),
        out_type=jax.ShapeDtypeStruct((_FLAT, 4 * _D), jnp.float32),
        scratch_types=[
            pltpu.VMEM((_PERW,), jnp.int32),
            [pltpu.VMEM((_CH, _D), jnp.float32) for _ in range(_NBUF)],
            [pltpu.SemaphoreType.DMA for _ in range(_NBUF)],
        ],
    )
    def _gather(tab_hbm, idx_hbm, out_hbm, idx_v, rows, sems):
        wid = lax.axis_index("s") * 2 + lax.axis_index("c")
        base = wid * _PERW
        pltpu.sync_copy(idx_hbm.at[pl.ds(base, _PERW)], idx_v)
        for j in range(_NBUF):
            pltpu.async_copy(
                tab_hbm.at[idx_v.at[pl.ds(j * _CH, _CH)]], rows[j], sems[j]
            )

        def outer(g, carry):
            for j in range(_NBUF):
                c = g * _NBUF + j
                pltpu.make_async_copy(
                    tab_hbm.at[idx_v.at[pl.ds(0, _CH)]], rows[j], sems[j]
                ).wait()
                pltpu.sync_copy(
                    rows[j],
                    out_hbm.at[pl.ds(base + c * _CH, _CH), pl.ds(0, _D)],
                )
                nc = c + _NBUF

                @pl.when(nc < _NCH)
                def _():
                    pltpu.async_copy(
                        tab_hbm.at[idx_v.at[pl.ds(nc * _CH, _CH)]],
                        rows[j],
                        sems[j],
                    )
            return carry

        lax.fori_loop(0, _NGRP, outer, 0)

    return _gather


_BB = 64


def _decode_body(r_ref, wf_ref, b_ref, o_ref):
    x = r_ref[...]
    lane = lax.broadcasted_iota(jnp.int32, x.shape, 1)
    xm = jnp.where(lane < _D, x, 0.0)
    y = (
        jnp.dot(xm, wf_ref[...], preferred_element_type=jnp.float32)
        + b_ref[...]
    )
    o_ref[...] = y.reshape(_BB, _L, _D)


def _decode(wide, Wfat, b2d):
    return pl.pallas_call(
        _decode_body,
        grid=(_B // _BB,),
        in_specs=[
            pl.BlockSpec((_BB * _L, 4 * _D), lambda i: (i, 0)),
            pl.BlockSpec((4 * _D, _D), lambda i: (0, 0)),
            pl.BlockSpec((1, _D), lambda i: (0, 0)),
        ],
        out_specs=pl.BlockSpec((_BB, _L, _D), lambda i: (i, 0, 0)),
        out_shape=jax.ShapeDtypeStruct((_B, _L, _D), jnp.float32),
    )(wide, Wfat, b2d)


def kernel(idx, table, W, b):
    wide = _build_gather()(table, idx.reshape(-1))
    Wfat = jnp.concatenate(
        [W.T, jnp.zeros((3 * _D, _D), jnp.float32)], axis=0
    )
    return _decode(wide, Wfat, b.reshape(1, _D))


# speedup vs baseline: 1.2735x; 1.0098x over previous
import functools

import jax
import jax.numpy as jnp
from jax import lax
from jax.experimental import pallas as pl
from jax.experimental.pallas import tpu as pltpu
from jax.experimental.pallas import tpu_sc as plsc

_V = 1_000_000
_D = 32
_B = 16384
_L = 50
_FLAT = _B * _L

_NW = 32
_CH = 640
_NBUF = 4


@functools.cache
def _build_gather(n_flat):
    perw = n_flat // _NW
    nch = perw // _CH
    ngrp = nch // _NBUF
    mesh = plsc.VectorSubcoreMesh(core_axis_name="c", subcore_axis_name="s")

    @functools.partial(
        pl.kernel,
        mesh=mesh,
        compiler_params=pltpu.CompilerParams(use_tc_tiling_on_sc=False),
        out_type=jax.ShapeDtypeStruct((n_flat, 4 * _D), jnp.float32),
        scratch_types=[
            pltpu.VMEM((perw,), jnp.int32),
            [pltpu.VMEM((_CH, _D), jnp.float32) for _ in range(_NBUF)],
            [pltpu.SemaphoreType.DMA for _ in range(_NBUF)],
        ],
    )
    def _gather(tab_hbm, idx_hbm, out_hbm, idx_v, rows, sems):
        wid = lax.axis_index("s") * 2 + lax.axis_index("c")
        base = wid * perw
        pltpu.sync_copy(idx_hbm.at[pl.ds(base, perw)], idx_v)
        for j in range(_NBUF):
            pltpu.async_copy(
                tab_hbm.at[idx_v.at[pl.ds(j * _CH, _CH)]], rows[j], sems[j]
            )

        def outer(g, carry):
            for j in range(_NBUF):
                c = g * _NBUF + j
                pltpu.make_async_copy(
                    tab_hbm.at[idx_v.at[pl.ds(0, _CH)]], rows[j], sems[j]
                ).wait()
                pltpu.sync_copy(
                    rows[j],
                    out_hbm.at[pl.ds(base + c * _CH, _CH), pl.ds(0, _D)],
                )
                nc = c + _NBUF

                @pl.when(nc < nch)
                def _():
                    pltpu.async_copy(
                        tab_hbm.at[idx_v.at[pl.ds(nc * _CH, _CH)]],
                        rows[j],
                        sems[j],
                    )
            return carry

        lax.fori_loop(0, ngrp, outer, 0)

    return _gather


_BB = 128


def _decode_body(r_ref, wf_ref, b_ref, o_ref):
    x = r_ref[...]
    lane = lax.broadcasted_iota(jnp.int32, x.shape, 1)
    xm = jnp.where(lane < _D, x, 0.0)
    y = (
        jnp.dot(xm, wf_ref[...], preferred_element_type=jnp.float32)
        + b_ref[...]
    )
    o_ref[...] = y.reshape(_BB, _L, _D)


def _decode(wide, Wfat, b2d):
    nb = wide.shape[0] // _L
    return pl.pallas_call(
        _decode_body,
        grid=(nb // _BB,),
        in_specs=[
            pl.BlockSpec((_BB * _L, 4 * _D), lambda i: (i, 0)),
            pl.BlockSpec((4 * _D, _D), lambda i: (0, 0)),
            pl.BlockSpec((1, _D), lambda i: (0, 0)),
        ],
        out_specs=pl.BlockSpec((_BB, _L, _D), lambda i: (i, 0, 0)),
        out_shape=jax.ShapeDtypeStruct((nb, _L, _D), jnp.float32),
    )(wide, Wfat, b2d)


def kernel(idx, table, W, b):
    idxf = idx.reshape(-1)
    half = _FLAT // 2
    Wfat = jnp.concatenate(
        [W.T, jnp.zeros((3 * _D, _D), jnp.float32)], axis=0
    )
    b2d = b.reshape(1, _D)
    g = _build_gather(half)
    wide_a = g(table, idxf[:half])
    wide_b = g(table, idxf[half:])
    out_a = _decode(wide_a, Wfat, b2d)
    out_b = _decode(wide_b, Wfat, b2d)
    return jnp.concatenate([out_a, out_b], axis=0)


# speedup vs baseline: 1.3271x; 1.0421x over previous
import functools

import jax
import jax.numpy as jnp
from jax import lax
from jax.experimental import pallas as pl
from jax.experimental.pallas import tpu as pltpu
from jax.experimental.pallas import tpu_sc as plsc

_V = 1_000_000
_D = 32
_B = 16384
_L = 50
_FLAT = _B * _L

_NW = 32
_PERW = _FLAT // _NW
_CH = 640
_NCH = _PERW // _CH
_NBUF = 4
_NGRP = _NCH // _NBUF


@functools.cache
def _build_gather():
    mesh = plsc.VectorSubcoreMesh(core_axis_name="c", subcore_axis_name="s")

    @functools.partial(
        pl.kernel,
        mesh=mesh,
        compiler_params=pltpu.CompilerParams(use_tc_tiling_on_sc=False),
        out_type=jax.ShapeDtypeStruct((_FLAT, 4 * _D), jnp.float32),
        scratch_types=[
            pltpu.VMEM((_PERW,), jnp.int32),
            [pltpu.VMEM((_CH, _D), jnp.float32) for _ in range(_NBUF)],
            [pltpu.SemaphoreType.DMA for _ in range(_NBUF)],
        ],
    )
    def _gather(tab_hbm, idx_hbm, out_hbm, idx_v, rows, sems):
        wid = lax.axis_index("s") * 2 + lax.axis_index("c")
        base = wid * _PERW
        pltpu.sync_copy(idx_hbm.at[pl.ds(base, _PERW)], idx_v)
        for j in range(_NBUF):
            pltpu.async_copy(
                tab_hbm.at[idx_v.at[pl.ds(j * _CH, _CH)]], rows[j], sems[j]
            )

        def outer(g, carry):
            for j in range(_NBUF):
                c = g * _NBUF + j
                pltpu.make_async_copy(
                    tab_hbm.at[idx_v.at[pl.ds(0, _CH)]], rows[j], sems[j]
                ).wait()
                pltpu.sync_copy(
                    rows[j],
                    out_hbm.at[pl.ds(base + c * _CH, _CH), pl.ds(0, _D)],
                )
                nc = c + _NBUF

                @pl.when(nc < _NCH)
                def _():
                    pltpu.async_copy(
                        tab_hbm.at[idx_v.at[pl.ds(nc * _CH, _CH)]],
                        rows[j],
                        sems[j],
                    )
            return carry

        lax.fori_loop(0, _NGRP, outer, 0)

    return _gather


_BB = 128


def _decode_body(r_ref, wf_ref, b_ref, o_ref):
    x = r_ref[...]
    lane = lax.broadcasted_iota(jnp.int32, x.shape, 1)
    xm = jnp.where(lane < _D, x, 0.0)
    y = (
        jnp.dot(xm, wf_ref[...], preferred_element_type=jnp.float32)
        + b_ref[...]
    )
    o_ref[...] = y.reshape(_BB, _L, _D)


def _decode(wide, Wfat, b2d):
    return pl.pallas_call(
        _decode_body,
        grid=(_B // _BB,),
        in_specs=[
            pl.BlockSpec((_BB * _L, 4 * _D), lambda i: (i, 0)),
            pl.BlockSpec((4 * _D, _D), lambda i: (0, 0)),
            pl.BlockSpec((1, _D), lambda i: (0, 0)),
        ],
        out_specs=pl.BlockSpec((_BB, _L, _D), lambda i: (i, 0, 0)),
        out_shape=jax.ShapeDtypeStruct((_B, _L, _D), jnp.float32),
    )(wide, Wfat, b2d)


def kernel(idx, table, W, b):
    wide = _build_gather()(table, idx.reshape(-1))
    Wfat = jnp.concatenate(
        [W.T, jnp.zeros((3 * _D, _D), jnp.float32)], axis=0
    )
    return _decode(wide, Wfat, b.reshape(1, _D))
